# Initial kernel scaffold; baseline (speedup 1.0000x reference)
import jax
import jax.numpy as jnp
from jax.experimental import pallas as pl


def kernel(x, edge_index, first_edge, candidate_edges, t, gin0_W1, gin0_b1, gin0_W2, gin0_b2, gin1_W1, gin1_b1, gin1_W2, gin1_b2, Wt1, bt1, Wt2, bt2, Wp1, bp1, Wp2, bp2):
    raise NotImplementedError("write your pallas kernel here")


# speedup vs baseline: 5.8626x; 5.8626x over previous
import functools

import jax
import jax.numpy as jnp
from jax import lax
from jax.experimental import pallas as pl
from jax.experimental.pallas import tpu as pltpu
from jax.experimental.pallas import tpu_sc as plsc

N = 10000
E = 320000
D = 128
C = 1024

NC = 2
NS = 16
NW = NC * NS
EPW = E // NW
CH = 80
NCH = EPW // CH
NP = 10240
RPS = NP // NS

GW = 72
GTOT = NW * GW


def _sc_mesh():
    return plsc.VectorSubcoreMesh(core_axis_name="c", subcore_axis_name="s")


def _segment_sum_partials(h, src_w, dst_w, zeros):
    @functools.partial(
        pl.kernel,
        out_type=jax.ShapeDtypeStruct((NC, NP, D), jnp.float32),
        mesh=_sc_mesh(),
        scratch_types=[
            pltpu.VMEM((NCH, CH), jnp.int32),
            pltpu.VMEM((NCH, CH), jnp.int32),
            pltpu.VMEM((CH, D), jnp.float32),
            pltpu.VMEM_SHARED((NP, D), jnp.float32),
        ],
    )
    def k(h_hbm, src_hbm, dst_hbm, z_hbm, out_hbm, src_v, dst_v, rows_v, agg_sh):
        cid = lax.axis_index("c")
        sid = lax.axis_index("s")
        wid = cid * NS + sid
        base = sid * RPS
        pltpu.sync_copy(z_hbm.at[pl.ds(base, RPS)], agg_sh.at[pl.ds(base, RPS)])
        pltpu.sync_copy(src_hbm.at[wid], src_v)
        pltpu.sync_copy(dst_hbm.at[wid], dst_v)
        plsc.subcore_barrier()

        @pl.loop(0, NCH)
        def _(i):
            pltpu.sync_copy(h_hbm.at[src_v.at[i]], rows_v)
            pltpu.sync_copy(rows_v, agg_sh.at[dst_v.at[i]], add=True)

        plsc.subcore_barrier()
        pltpu.sync_copy(agg_sh.at[pl.ds(base, RPS)],
                        out_hbm.at[cid, pl.ds(base, RPS)])

    return k(h, src_w, dst_w, zeros)


RB = 1000


def _gin_mlp(x, p, W1, b1, W2, b2):
    def body(x_ref, p_ref, w1_ref, b1_ref, w2_ref, b2_ref, o_ref):
        dot = lambda a, b: lax.dot_general(
            a, b, (((1,), (0,)), ((), ())),
            precision=lax.Precision.HIGHEST,
            preferred_element_type=jnp.float32)
        s = x_ref[...] + p_ref[0] + p_ref[1]
        a = jnp.maximum(dot(s, w1_ref[...]) + b1_ref[...], 0.0)
        o_ref[...] = dot(a, w2_ref[...]) + b2_ref[...]

    return pl.pallas_call(
        body,
        grid=(N // RB,),
        in_specs=[
            pl.BlockSpec((RB, D), lambda i: (i, 0)),
            pl.BlockSpec((NC, RB, D), lambda i: (0, i, 0)),
            pl.BlockSpec((D, D), lambda i: (0, 0)),
            pl.BlockSpec((1, D), lambda i: (0, 0)),
            pl.BlockSpec((D, D), lambda i: (0, 0)),
            pl.BlockSpec((1, D), lambda i: (0, 0)),
        ],
        out_specs=pl.BlockSpec((RB, D), lambda i: (i, 0)),
        out_shape=jax.ShapeDtypeStruct((N, D), jnp.float32),
    )(x, p, W1, b1.reshape(1, D), W2, b2.reshape(1, D))


def _gather_rows(h, idx_pad):
    @functools.partial(
        pl.kernel,
        out_type=jax.ShapeDtypeStruct((GTOT, D), jnp.float32),
        mesh=_sc_mesh(),
        scratch_types=[
            pltpu.VMEM((GW,), jnp.int32),
            pltpu.VMEM((GW, D), jnp.float32),
        ],
    )
    def k(h_hbm, idx_hbm, out_hbm, idx_v, rows_v):
        cid = lax.axis_index("c")
        sid = lax.axis_index("s")
        base = (cid * NS + sid) * GW
        pltpu.sync_copy(idx_hbm.at[pl.ds(base, GW)], idx_v)
        pltpu.sync_copy(h_hbm.at[idx_v], rows_v)
        pltpu.sync_copy(rows_v, out_hbm.at[pl.ds(base, GW)])

    return k(h, idx_pad)


def _score(g, t_row, Wt1, bt1, Wt2, bt2, Wp1, bp1, Wp2, bp2):
    def body(g_ref, t_ref, wt1_ref, bt1_ref, wt2_ref, bt2_ref,
             wp1_ref, bp1_ref, wp2_ref, o_ref):
        dot = lambda a, b: lax.dot_general(
            a, b, (((1,), (0,)), ((), ())),
            precision=lax.Precision.HIGHEST,
            preferred_element_type=jnp.float32)
        hu = g_ref[0:1, :]
        hv = g_ref[1:2, :]
        t_emb = dot(jnp.maximum(t_ref[...] * wt1_ref[...] + bt1_ref[...], 0.0),
                    wt2_ref[...]) + bt2_ref[...]
        bias = (dot(hu + hv, wp1_ref[0:D, :])
                + dot(jnp.abs(hu - hv), wp1_ref[D:2 * D, :])
                + dot(t_emb, wp1_ref[4 * D:5 * D, :])
                + bp1_ref[...])
        cu = g_ref[2:2 + C, :]
        cv = g_ref[2 + C:2 + 2 * C, :]
        acts = jnp.maximum(dot(cu + cv, wp1_ref[2 * D:3 * D, :])
                           + dot(jnp.abs(cu - cv), wp1_ref[3 * D:4 * D, :])
                           + bias, 0.0)
        o_ref[...] = jnp.sum(acts * wp2_ref[...], axis=1, keepdims=True)

    out = pl.pallas_call(
        body,
        out_shape=jax.ShapeDtypeStruct((C, 1), jnp.float32),
    )(g, t_row, Wt1, bt1.reshape(1, D), Wt2, bt2.reshape(1, D),
      Wp1, bp1.reshape(1, D), Wp2.reshape(1, D))
    return out[:, 0] + bp2[0]


def kernel(x, edge_index, first_edge, candidate_edges, t,
           gin0_W1, gin0_b1, gin0_W2, gin0_b2,
           gin1_W1, gin1_b1, gin1_W2, gin1_b2,
           Wt1, bt1, Wt2, bt2, Wp1, bp1, Wp2, bp2):
    src_w = edge_index[0].reshape(NW, NCH, CH)
    dst_w = edge_index[1].reshape(NW, NCH, CH)
    zeros = jnp.zeros((NP, D), jnp.float32)

    p0 = _segment_sum_partials(x, src_w, dst_w, zeros)
    h1 = _gin_mlp(x, p0, gin0_W1, gin0_b1, gin0_W2, gin0_b2)
    p1 = _segment_sum_partials(h1, src_w, dst_w, zeros)
    h2 = _gin_mlp(h1, p1, gin1_W1, gin1_b1, gin1_W2, gin1_b2)

    idx_pad = jnp.concatenate([
        first_edge.astype(jnp.int32),
        candidate_edges[:, 0].astype(jnp.int32),
        candidate_edges[:, 1].astype(jnp.int32),
        jnp.zeros((GTOT - 2 - 2 * C,), jnp.int32),
    ])
    g = _gather_rows(h2, idx_pad)

    t_row = jnp.full((1, D), 1.0, jnp.float32) * jnp.asarray(t, jnp.float32)
    return _score(g, t_row, Wt1, bt1, Wt2, bt2, Wp1, bp1, Wp2, bp2)
